# baseline (speedup 1.0000x reference)
import functools

import jax
import jax.numpy as jnp
from jax.experimental import pallas as pl
from jax.experimental.pallas import tpu as pltpu

B = 1024
R = 256


def _cumsum_kernel(x_ref, o_ref, carry_ref, *, blk, grp):
    s = pl.program_id(1)

    @pl.when(s == 0)
    def _():
        carry_ref[...] = jnp.zeros_like(carry_ref)

    tri = jnp.tril(jnp.ones((grp, grp), dtype=jnp.float32)).astype(jnp.bfloat16)
    carry = carry_ref[...]
    for g in range(blk // grp):
        xg = x_ref[0, g * grp : (g + 1) * grp, :]
        local = jax.lax.dot(
            tri, xg.astype(jnp.bfloat16), preferred_element_type=jnp.float32
        )
        out = local + carry
        o_ref[0, g * grp : (g + 1) * grp, :] = out
        carry = out[grp - 1 :, :]
    carry_ref[...] = carry


def kernel(x):
    batch, seq, feat = x.shape
    grid = (batch, seq // B)
    f = pl.pallas_call(
        functools.partial(_cumsum_kernel, blk=B, grp=R),
        grid=grid,
        in_specs=[pl.BlockSpec((1, B, feat), lambda b, s: (b, s, 0))],
        out_specs=pl.BlockSpec((1, B, feat), lambda b, s: (b, s, 0)),
        out_shape=jax.ShapeDtypeStruct(x.shape, x.dtype),
        scratch_shapes=[pltpu.VMEM((1, feat), jnp.float32)],
        compiler_params=pltpu.CompilerParams(
            dimension_semantics=("parallel", "arbitrary"),
        ),
    )
    return f(x)
